# Initial kernel scaffold; baseline (speedup 1.0000x reference)
import functools

import jax
import jax.numpy as jnp
from jax import lax
from jax.experimental import pallas as pl
from jax.experimental.pallas import tpu as pltpu
from jax.experimental.pallas import tpu_sc as plsc

SEQ = 200
B = 1024
V = 100000
D = 64
H = 64

NC = 2
NS = 16
NW = NC * NS
N = SEQ * B
PER_W = N // NW
CHUNK = 128
GROUP = 10
ROWS = CHUNK * GROUP
NGROUP = PER_W // ROWS


def _sc_gather(table, idx2d):
    mesh = plsc.VectorSubcoreMesh(core_axis_name="c", subcore_axis_name="s")

    @functools.partial(
        pl.kernel,
        out_type=jax.ShapeDtypeStruct((N, D), jnp.float32),
        mesh=mesh,
        scratch_types=[
            pltpu.VMEM((GROUP, CHUNK), jnp.int32),
            pltpu.VMEM((ROWS, D), jnp.float32),
            pltpu.SemaphoreType.DMA,
        ],
    )
    def k(table_hbm, idx_hbm, out_hbm, idx_v, rows_v, sem):
        wid = lax.axis_index("s") * NC + lax.axis_index("c")

        @pl.loop(0, NGROUP)
        def group(g):
            row0 = wid * PER_W + g * ROWS
            pltpu.sync_copy(idx_hbm.at[pl.ds(row0 // CHUNK, GROUP)], idx_v)
            copies = [
                pltpu.async_copy(
                    table_hbm.at[idx_v.at[j]],
                    rows_v.at[pl.ds(j * CHUNK, CHUNK)],
                    sem,
                )
                for j in range(GROUP)
            ]
            for c in copies:
                c.wait()
            pltpu.sync_copy(rows_v, out_hbm.at[pl.ds(row0, ROWS)])

    return k(table, idx2d)


def _tc_gru(emb, w_ih_t, w_hh_t, b_i, b_h, interpret=False):
    def body(emb_ref, wih_ref, whh_ref, bi_ref, bh_ref, out_ref, h_ref):
        t = pl.program_id(0)

        @pl.when(t == 0)
        def _():
            h_ref[...] = jnp.zeros_like(h_ref)

        xt = emb_ref[0]
        h = h_ref[...]
        gi = jnp.dot(xt, wih_ref[...], preferred_element_type=jnp.float32)
        gi = gi + bi_ref[...]
        gh = jnp.dot(h, whh_ref[...], preferred_element_type=jnp.float32)
        gh = gh + bh_ref[...]
        rz = jax.nn.sigmoid(gi[:, : 2 * H] + gh[:, : 2 * H])
        r = rz[:, :H]
        z = rz[:, H:]
        n = jnp.tanh(gi[:, 2 * H :] + r * gh[:, 2 * H :])
        h_new = (1.0 - z) * n + z * h
        h_ref[...] = h_new

        @pl.when(t == SEQ - 1)
        def _():
            out_ref[...] = h_new

    return pl.pallas_call(
        body,
        grid=(SEQ,),
        in_specs=[
            pl.BlockSpec((1, B, D), lambda t: (t, 0, 0)),
            pl.BlockSpec((D, 3 * H), lambda t: (0, 0)),
            pl.BlockSpec((H, 3 * H), lambda t: (0, 0)),
            pl.BlockSpec((1, 3 * H), lambda t: (0, 0)),
            pl.BlockSpec((1, 3 * H), lambda t: (0, 0)),
        ],
        out_specs=pl.BlockSpec((B, H), lambda t: (0, 0)),
        out_shape=jax.ShapeDtypeStruct((B, H), jnp.float32),
        scratch_shapes=[pltpu.VMEM((B, H), jnp.float32)],
        interpret=interpret,
    )(emb, w_ih_t, w_hh_t, b_i, b_h)


def kernel(x, table, W_ih, W_hh, b_ih, b_hh):
    idx2d = x.reshape(N // CHUNK, CHUNK).astype(jnp.int32)
    emb = _sc_gather(table, idx2d)
    hn = _tc_gru(
        emb.reshape(SEQ, B, D),
        W_ih.T,
        W_hh.T,
        b_ih.reshape(1, 3 * H),
        b_hh.reshape(1, 3 * H),
    )
    return hn[None]


# speedup vs baseline: 3.0315x; 3.0315x over previous
import functools

import jax
import jax.numpy as jnp
from jax import lax
from jax.experimental import pallas as pl
from jax.experimental.pallas import tpu as pltpu
from jax.experimental.pallas import tpu_sc as plsc

SEQ = 200
B = 1024
V = 100000
D = 64
H = 64

NC = 2
NS = 16
NW = NC * NS
N = SEQ * B
PER_W = N // NW
CHUNK = 128
GROUP = 10
ROWS = CHUNK * GROUP
NGROUP = PER_W // ROWS


def _sc_gather(table, idx):
    mesh = plsc.VectorSubcoreMesh(core_axis_name="c", subcore_axis_name="s")

    @functools.partial(
        pl.kernel,
        out_type=jax.ShapeDtypeStruct((N, D), jnp.float32),
        mesh=mesh,
        scratch_types=[
            pltpu.VMEM((ROWS,), jnp.int32),
            pltpu.VMEM((ROWS, D), jnp.float32),
            pltpu.SemaphoreType.DMA,
        ],
        compiler_params=pltpu.CompilerParams(use_tc_tiling_on_sc=False),
    )
    def k(table_hbm, idx_hbm, out_hbm, idx_v, rows_v, sem):
        wid = lax.axis_index("s") * NC + lax.axis_index("c")

        @pl.loop(0, NGROUP)
        def group(g):
            row0 = wid * PER_W + g * ROWS
            pltpu.sync_copy(idx_hbm.at[pl.ds(row0, ROWS)], idx_v)
            copies = [
                pltpu.async_copy(
                    table_hbm.at[idx_v.at[pl.ds(j * CHUNK, CHUNK)]],
                    rows_v.at[pl.ds(j * CHUNK, CHUNK)],
                    sem,
                )
                for j in range(GROUP)
            ]
            for c in copies:
                c.wait()
            pltpu.sync_copy(rows_v, out_hbm.at[pl.ds(row0, ROWS)])

    return k(table, idx)


def _tc_gru(emb, w_ih_t, w_hh_t, b_i, b_h, interpret=False):
    def body(emb_ref, wih_ref, whh_ref, bi_ref, bh_ref, out_ref, h_ref):
        t = pl.program_id(0)

        @pl.when(t == 0)
        def _():
            h_ref[...] = jnp.zeros_like(h_ref)

        xt = emb_ref[0]
        h = h_ref[...]
        gi = jnp.dot(xt, wih_ref[...], preferred_element_type=jnp.float32)
        gi = gi + bi_ref[...]
        gh = jnp.dot(h, whh_ref[...], preferred_element_type=jnp.float32)
        gh = gh + bh_ref[...]
        rz = jax.nn.sigmoid(gi[:, : 2 * H] + gh[:, : 2 * H])
        r = rz[:, :H]
        z = rz[:, H:]
        n = jnp.tanh(gi[:, 2 * H :] + r * gh[:, 2 * H :])
        h_new = (1.0 - z) * n + z * h
        h_ref[...] = h_new

        @pl.when(t == SEQ - 1)
        def _():
            out_ref[...] = h_new

    return pl.pallas_call(
        body,
        grid=(SEQ,),
        in_specs=[
            pl.BlockSpec((1, B, D), lambda t: (t, 0, 0)),
            pl.BlockSpec((D, 3 * H), lambda t: (0, 0)),
            pl.BlockSpec((H, 3 * H), lambda t: (0, 0)),
            pl.BlockSpec((1, 3 * H), lambda t: (0, 0)),
            pl.BlockSpec((1, 3 * H), lambda t: (0, 0)),
        ],
        out_specs=pl.BlockSpec((B, H), lambda t: (0, 0)),
        out_shape=jax.ShapeDtypeStruct((B, H), jnp.float32),
        scratch_shapes=[pltpu.VMEM((B, H), jnp.float32)],
        interpret=interpret,
    )(emb, w_ih_t, w_hh_t, b_i, b_h)


def kernel(x, table, W_ih, W_hh, b_ih, b_hh):
    idx = x.reshape(N).astype(jnp.int32)
    emb = _sc_gather(table, idx)
    hn = _tc_gru(
        emb.reshape(SEQ, B, D),
        W_ih.T,
        W_hh.T,
        b_ih.reshape(1, 3 * H),
        b_hh.reshape(1, 3 * H),
    )
    return hn[None]


# speedup vs baseline: 4.2782x; 1.4112x over previous
import functools

import jax
import jax.numpy as jnp
from jax import lax
from jax.experimental import pallas as pl
from jax.experimental.pallas import tpu as pltpu
from jax.experimental.pallas import tpu_sc as plsc

SEQ = 200
B = 1024
V = 100000
D = 64
H = 64

NC = 2
NS = 16
NW = NC * NS
N = SEQ * B
PER_W = N // NW
CHUNK = 128
GROUP = 10
ROWS = CHUNK * GROUP
NGROUP = PER_W // ROWS


def _sc_gather(table, idx):
    mesh = plsc.VectorSubcoreMesh(core_axis_name="c", subcore_axis_name="s")

    @functools.partial(
        pl.kernel,
        out_type=jax.ShapeDtypeStruct((N, D), jnp.float32),
        mesh=mesh,
        scratch_types=[
            pltpu.VMEM((ROWS,), jnp.int32),
            pltpu.VMEM((ROWS, D), jnp.float32),
            pltpu.SemaphoreType.DMA,
        ],
        compiler_params=pltpu.CompilerParams(use_tc_tiling_on_sc=False),
    )
    def k(table_hbm, idx_hbm, out_hbm, idx_v, rows_v, sem):
        wid = lax.axis_index("s") * NC + lax.axis_index("c")

        @pl.loop(0, NGROUP)
        def group(g):
            row0 = wid * PER_W + g * ROWS
            pltpu.sync_copy(idx_hbm.at[pl.ds(row0, ROWS)], idx_v)
            copies = [
                pltpu.async_copy(
                    table_hbm.at[idx_v.at[pl.ds(j * CHUNK, CHUNK)]],
                    rows_v.at[pl.ds(j * CHUNK, CHUNK)],
                    sem,
                )
                for j in range(GROUP)
            ]
            for c in copies:
                c.wait()
            pltpu.sync_copy(rows_v, out_hbm.at[pl.ds(row0, ROWS)])

    return k(table, idx)


T_BLK = 8
N_TBLK = SEQ // T_BLK


def _tc_gru(emb, w_ih, w_hh, brz, bin_, bhn, interpret=False):
    rhs_t = (((1,), (1,)), ((), ()))

    def body(emb_ref, wih_ref, whh_ref, brz_ref, bin_ref, bhn_ref,
             out_ref, h_ref):
        t = pl.program_id(0)

        @pl.when(t == 0)
        def _():
            h_ref[...] = jnp.zeros_like(h_ref)

        wih = wih_ref[...]
        whh = whh_ref[...]
        for i in range(T_BLK):
            h = h_ref[...]
            gi = jax.lax.dot_general(
                wih, emb_ref[i], rhs_t, preferred_element_type=jnp.float32)
            gh = jnp.dot(whh, h, preferred_element_type=jnp.float32)
            rz = jax.nn.sigmoid(gi[: 2 * H] + gh[: 2 * H] + brz_ref[...])
            r = rz[:H]
            z = rz[H:]
            n = jnp.tanh(gi[2 * H :] + bin_ref[...]
                         + r * (gh[2 * H :] + bhn_ref[...]))
            h_new = n + z * (h - n)
            h_ref[...] = h_new

        @pl.when(t == N_TBLK - 1)
        def _():
            out_ref[...] = h_ref[...]

    return pl.pallas_call(
        body,
        grid=(N_TBLK,),
        in_specs=[
            pl.BlockSpec((T_BLK, B, D), lambda t: (t, 0, 0)),
            pl.BlockSpec((3 * H, D), lambda t: (0, 0)),
            pl.BlockSpec((3 * H, H), lambda t: (0, 0)),
            pl.BlockSpec((2 * H, B), lambda t: (0, 0)),
            pl.BlockSpec((H, B), lambda t: (0, 0)),
            pl.BlockSpec((H, B), lambda t: (0, 0)),
        ],
        out_specs=pl.BlockSpec((H, B), lambda t: (0, 0)),
        out_shape=jax.ShapeDtypeStruct((H, B), jnp.float32),
        scratch_shapes=[pltpu.VMEM((H, B), jnp.float32)],
        interpret=interpret,
    )(emb, w_ih, w_hh, brz, bin_, bhn)


def kernel(x, table, W_ih, W_hh, b_ih, b_hh):
    idx = x.reshape(N).astype(jnp.int32)
    emb = _sc_gather(table, idx)
    brz = jnp.broadcast_to((b_ih[: 2 * H] + b_hh[: 2 * H])[:, None], (2 * H, B))
    bin_ = jnp.broadcast_to(b_ih[2 * H :][:, None], (H, B))
    bhn = jnp.broadcast_to(b_hh[2 * H :][:, None], (H, B))
    hn_t = _tc_gru(emb.reshape(SEQ, B, D), W_ih, W_hh, brz, bin_, bhn)
    return hn_t.T[None]
